# baseline (speedup 1.0000x reference)
import functools

import jax
import jax.numpy as jnp
from jax import lax
from jax.experimental import pallas as pl
from jax.experimental.pallas import tpu as pltpu
from jax.experimental.pallas import tpu_sc as plsc

B = 4
N_LOW = 8192
N_HIGH = 16384
C_IN = 128
C_OUT = 64
J = 9
NNZ = 3 * N_HIGH

NC, NS, LANES = 2, 16, 16
NW = NC * NS
ROWS_PER_W = N_HIGH // NW
CH = 128
GCH = 64
N_GCH = ROWS_PER_W // GCH

_mesh = lambda: plsc.VectorSubcoreMesh(core_axis_name="c", subcore_axis_name="s")


NSEG = 4
QHALF = N_HIGH // NSEG
SLAB = QHALF // NS


def _pool_body(x2_hbm, cols_hbm, rows_hbm, vals_hbm, bounds_hbm, out_hbm,
               boundsv, colv, rowv, valv, idxb, xg, zbuf, shared, sem):
    cid = lax.axis_index("c")
    sid = lax.axis_index("s")
    pltpu.sync_copy(bounds_hbm, boundsv)

    def zb_body(r, _):
        for cb in range(C_IN // LANES):
            zbuf[r, pl.ds(cb * LANES, LANES)] = jnp.zeros((LANES,), jnp.float32)
        return 0
    lax.fori_loop(0, SLAB, zb_body, 0)

    def bp_body(t, _):
        b = t // 2
        seg = cid * 2 + (t % 2)
        rbase = seg * QHALF
        s = boundsv[pl.ds(seg * 8, LANES)][0]
        e = boundsv[pl.ds(seg * 8 + 8, LANES)][0]
        a = (s // 8) * 8
        nch = (e - a + CH - 1) // CH
        my_nch = (nch - sid + NS - 1) // NS

        pltpu.sync_copy(zbuf, shared.at[pl.ds(sid * SLAB, SLAB)])
        plsc.subcore_barrier()

        col_off = b * N_LOW

        def chunk_body(i, _):
            off = a + (sid + i * NS) * CH
            pltpu.sync_copy(cols_hbm.at[pl.ds(off, CH)], colv)
            pltpu.sync_copy(rows_hbm.at[pl.ds(off, CH)], rowv)
            pltpu.sync_copy(vals_hbm.at[pl.ds(off, CH)],
                            valv.at[pl.ds(0, CH)])
            for q in range(CH // LANES):
                sl = pl.ds(q * LANES, LANES)
                colv[sl] = colv[sl] + col_off
                idxb[sl] = jnp.clip(rowv[sl] - rbase, 0, QHALF - 1)
            pltpu.async_copy(x2_hbm.at[colv], xg, sem).wait()

            def r_body(r, _):
                kk = off + r
                v = valv[pl.ds(r, LANES)][0]
                v = jnp.where((kk >= s) & (kk < e), v, 0.0)
                vv = jnp.full((LANES,), v, jnp.float32)
                for cb in range(C_IN // LANES):
                    sl = pl.ds(cb * LANES, LANES)
                    xg[r, sl] = xg[r, sl] * vv
                return 0
            lax.fori_loop(0, CH, r_body, 0)

            pltpu.sync_copy(xg, shared.at[idxb], add=True)
            return 0
        lax.fori_loop(0, my_nch, chunk_body, 0)

        plsc.subcore_barrier()
        pltpu.sync_copy(shared.at[pl.ds(sid * SLAB, SLAB)],
                        out_hbm.at[b, pl.ds(rbase + sid * SLAB, SLAB)])
        plsc.subcore_barrier()
        return 0
    lax.fori_loop(0, B * 2, bp_body, 0)


def _pool(x2, cols, rows, vals, bounds):
    k = pl.kernel(
        _pool_body,
        out_type=jax.ShapeDtypeStruct((B, N_HIGH, C_IN), jnp.float32),
        mesh=_mesh(),
        scratch_types=[
            pltpu.VMEM((NW + LANES + 8,), jnp.int32),
            pltpu.VMEM((CH,), jnp.int32),
            pltpu.VMEM((CH,), jnp.int32),
            pltpu.VMEM((CH + LANES,), jnp.float32),
            pltpu.VMEM((CH,), jnp.int32),
            pltpu.VMEM((CH, C_IN), jnp.float32),
            pltpu.VMEM((SLAB, C_IN), jnp.float32),
            pltpu.VMEM_SHARED((QHALF, C_IN), jnp.float32),
            pltpu.SemaphoreType.DMA,
        ],
    )
    return k(x2, cols, rows, vals, bounds)


NP = 5
MM_BT = 2048


def _mm_body(p_ref, w_ref, *o_refs):
    res = lax.dot_general(
        p_ref[...], w_ref[...], (((1,), (0,)), ((), ())),
        preferred_element_type=jnp.float32)
    for p5 in range(NP):
        o_refs[p5][...] = res[:, p5 * C_IN:(p5 + 1) * C_IN].astype(jnp.bfloat16)


def _matmul(pooled2, w640):
    return pl.pallas_call(
        _mm_body,
        grid=(B * N_HIGH // MM_BT,),
        in_specs=[
            pl.BlockSpec((MM_BT, C_IN), lambda t: (t, 0)),
            pl.BlockSpec((C_IN, NP * C_IN), lambda t: (0, 0)),
        ],
        out_specs=[pl.BlockSpec((MM_BT, C_IN), lambda t: (t, 0))] * NP,
        out_shape=[jax.ShapeDtypeStruct((B * N_HIGH, C_IN), jnp.bfloat16)] * NP,
    )(pooled2, w640)


N_CHUNKS = B * N_GCH


def _spiral_body(h0, h1, h2, h3, h4, idxt_hbm, bias_hbm, out_hbm,
                 biasv, idxfull, idxv, gbuf, ybuf, sem):
    tables = (h0, h1, h2, h3, h4)
    wid = lax.axis_index("s") * NC + lax.axis_index("c")
    nb = wid * ROWS_PER_W
    pltpu.sync_copy(bias_hbm, biasv)
    for j in range(J):
        pltpu.sync_copy(idxt_hbm.at[pl.ds(j * N_HIGH + nb, ROWS_PER_W)],
                        idxfull.at[pl.ds(j * ROWS_PER_W, ROWS_PER_W)])

    def load_idx(t, par):
        b = t // N_GCH
        c = t % N_GCH
        off = b * N_HIGH
        for j in range(J):
            for q in range(GCH // LANES):
                dsl = pl.ds(par * J * GCH + j * GCH + q * LANES, LANES)
                ssl = pl.ds(j * ROWS_PER_W + c * GCH + q * LANES, LANES)
                idxv[dsl] = idxfull[ssl] + off

    def fire(par):
        for j in range(J):
            pltpu.async_copy(
                tables[j // 2].at[
                    idxv.at[pl.ds(par * J * GCH + j * GCH, GCH)]],
                gbuf.at[par, j], sem)

    def drain(par):
        for j in range(J):
            pltpu.make_async_copy(
                tables[j // 2].at[
                    idxv.at[pl.ds(par * J * GCH + j * GCH, GCH)]],
                gbuf.at[par, j], sem).wait()

    def accum_store(t, par):
        b = t // N_GCH
        c = t % N_GCH

        def r_body(r, _):
            acc = [biasv[pl.ds(cb * LANES, LANES)]
                   for cb in range(C_OUT // LANES)]
            for j in range(J):
                for g in range(2):
                    hsl = pl.ds((j % 2) * C_OUT + g * 32, 32)
                    lo, hi = plsc.unpack(gbuf[par, j, r, hsl],
                                         format=plsc.PackFormat.INTERLEAVED)
                    acc[2 * g] = acc[2 * g] + lo
                    acc[2 * g + 1] = acc[2 * g + 1] + hi
            for cb in range(C_OUT // LANES):
                ybuf[r, pl.ds(cb * LANES, LANES)] = jnp.maximum(acc[cb], 0.0)
            return 0
        lax.fori_loop(0, GCH, r_body, 0)

        pltpu.sync_copy(ybuf, out_hbm.at[b, pl.ds(nb + c * GCH, GCH)])

    load_idx(0, 0)
    fire(0)

    def t2_body(t2, _):
        t = 2 * t2
        drain(0)
        load_idx(t + 1, 1)
        fire(1)
        accum_store(t, 0)
        drain(1)

        @pl.when(t2 < N_CHUNKS // 2 - 1)
        def _():
            load_idx(t + 2, 0)
            fire(0)
        accum_store(t + 1, 1)
        return 0
    lax.fori_loop(0, N_CHUNKS // 2, t2_body, 0)


def _spiral(hs, idxt, bias):
    k = pl.kernel(
        _spiral_body,
        out_type=jax.ShapeDtypeStruct((B, N_HIGH, C_OUT), jnp.float32),
        mesh=_mesh(),
        scratch_types=[
            pltpu.VMEM((C_OUT,), jnp.float32),
            pltpu.VMEM((J * ROWS_PER_W,), jnp.int32),
            pltpu.VMEM((2 * J * GCH,), jnp.int32),
            pltpu.VMEM((2, J, GCH, C_IN), jnp.bfloat16),
            pltpu.VMEM((GCH, C_OUT), jnp.float32),
            pltpu.SemaphoreType.DMA,
        ],
        compiler_params=pltpu.CompilerParams(use_tc_tiling_on_sc=False,
                                             needs_layout_passes=False),
    )
    return k(*hs, idxt, bias)


def kernel(x, up_rows, up_cols, up_vals, indices,
           W1, b1, W_d3, b_d3, W_2d3, b_2d3, W_full, b_full):
    wf = W_full.reshape(C_OUT // 2, J, C_IN).transpose(1, 0, 2)
    w2 = W_2d3.reshape(C_OUT // 4, 6, C_IN).transpose(1, 0, 2)
    w2 = jnp.concatenate([w2, jnp.zeros((3, C_OUT // 4, C_IN), jnp.float32)], 0)
    w3 = W_d3.reshape(C_OUT // 4, 3, C_IN).transpose(1, 0, 2)
    w3 = jnp.concatenate([w3, jnp.zeros((6, C_OUT // 4, C_IN), jnp.float32)], 0)
    wcat = jnp.concatenate([wf, w2, w3], axis=1)
    wcat = wcat.at[0].add(W1)
    bias = jnp.concatenate([b_full, b_2d3, b_d3]) + b1

    edges = jnp.arange(0, N_HIGH + 1, ROWS_PER_W, dtype=jnp.int32)
    bounds = jnp.searchsorted(up_rows, edges).astype(jnp.int32)
    bounds = jnp.concatenate(
        [bounds, jnp.zeros((NW + LANES + 8 - (NW + 1),), jnp.int32)])
    cols_p = jnp.concatenate([up_cols.astype(jnp.int32),
                              jnp.zeros((CH,), jnp.int32)])
    rows_p = jnp.concatenate([up_rows.astype(jnp.int32),
                              jnp.full((CH,), N_HIGH, jnp.int32)])
    vals_p = jnp.concatenate([up_vals, jnp.zeros((CH,), jnp.float32)])

    w640 = jnp.pad(wcat.transpose(2, 0, 1).reshape(C_IN, J * C_OUT),
                   ((0, 0), (0, NP * C_IN - J * C_OUT)))
    g16 = jnp.arange(16, dtype=jnp.int32)
    perm32 = jnp.stack([g16, g16 + 16], axis=1).reshape(32)
    perm = (jnp.arange(NP * C_IN // 32, dtype=jnp.int32)[:, None] * 32
            + perm32[None, :]).reshape(NP * C_IN)
    w640 = w640[:, perm]

    x2 = x.reshape(B * N_LOW, C_IN)
    pooled = _pool(x2, cols_p, rows_p, vals_p, bounds)
    pooled2 = pooled.reshape(B * N_HIGH, C_IN)
    hs = _matmul(pooled2, w640)
    idxt = indices.astype(jnp.int32).T.reshape(J * N_HIGH)
    return _spiral(hs, idxt, bias)


# speedup vs baseline: 1.2878x; 1.2878x over previous
import functools

import jax
import jax.numpy as jnp
from jax import lax
from jax.experimental import pallas as pl
from jax.experimental.pallas import tpu as pltpu
from jax.experimental.pallas import tpu_sc as plsc

B = 4
N_LOW = 8192
N_HIGH = 16384
C_IN = 128
C_OUT = 64
J = 9
NNZ = 3 * N_HIGH

NC, NS, LANES = 2, 16, 16
NW = NC * NS
ROWS_PER_W = N_HIGH // NW
CH = 128
GCH = 32
N_GCH = ROWS_PER_W // GCH

_mesh = lambda: plsc.VectorSubcoreMesh(core_axis_name="c", subcore_axis_name="s")


NSEG = 4
QHALF = N_HIGH // NSEG
SLAB = QHALF // NS


def _pool_body(x2_hbm, cols_hbm, rows_hbm, vals_hbm, bounds_hbm, out_hbm,
               boundsv, colv, rowv, valv, idxb, xg, zbuf, shared, sem):
    cid = lax.axis_index("c")
    sid = lax.axis_index("s")
    pltpu.sync_copy(bounds_hbm, boundsv)

    def zb_body(r, _):
        for cb in range(C_IN // LANES):
            zbuf[r, pl.ds(cb * LANES, LANES)] = jnp.zeros((LANES,), jnp.float32)
        return 0
    lax.fori_loop(0, SLAB, zb_body, 0)

    def bp_body(t, _):
        b = t // 2
        seg = cid * 2 + (t % 2)
        rbase = seg * QHALF
        s = boundsv[pl.ds(seg * 8, LANES)][0]
        e = boundsv[pl.ds(seg * 8 + 8, LANES)][0]
        a = (s // 8) * 8
        nch = (e - a + CH - 1) // CH
        my_nch = (nch - sid + NS - 1) // NS

        pltpu.sync_copy(zbuf, shared.at[pl.ds(sid * SLAB, SLAB)])
        plsc.subcore_barrier()

        col_off = b * N_LOW

        def chunk_body(i, _):
            off = a + (sid + i * NS) * CH
            pltpu.sync_copy(cols_hbm.at[pl.ds(off, CH)], colv)
            pltpu.sync_copy(rows_hbm.at[pl.ds(off, CH)], rowv)
            pltpu.sync_copy(vals_hbm.at[pl.ds(off, CH)],
                            valv.at[pl.ds(0, CH)])
            for q in range(CH // LANES):
                sl = pl.ds(q * LANES, LANES)
                colv[sl] = colv[sl] + col_off
                idxb[sl] = jnp.clip(rowv[sl] - rbase, 0, QHALF - 1)
            pltpu.async_copy(x2_hbm.at[colv], xg, sem).wait()

            def r_body(r, _):
                kk = off + r
                v = valv[pl.ds(r, LANES)][0]
                v = jnp.where((kk >= s) & (kk < e), v, 0.0)
                vv = jnp.full((LANES,), v, jnp.float32)
                for cb in range(C_IN // LANES):
                    sl = pl.ds(cb * LANES, LANES)
                    xg[r, sl] = xg[r, sl] * vv
                return 0
            lax.fori_loop(0, CH, r_body, 0)

            pltpu.sync_copy(xg, shared.at[idxb], add=True)
            return 0
        lax.fori_loop(0, my_nch, chunk_body, 0)

        plsc.subcore_barrier()
        pltpu.sync_copy(shared.at[pl.ds(sid * SLAB, SLAB)],
                        out_hbm.at[b, pl.ds(rbase + sid * SLAB, SLAB)])
        plsc.subcore_barrier()
        return 0
    lax.fori_loop(0, B * 2, bp_body, 0)


def _pool(x2, cols, rows, vals, bounds):
    k = pl.kernel(
        _pool_body,
        out_type=jax.ShapeDtypeStruct((B, N_HIGH, C_IN), jnp.float32),
        mesh=_mesh(),
        scratch_types=[
            pltpu.VMEM((NW + LANES + 8,), jnp.int32),
            pltpu.VMEM((CH,), jnp.int32),
            pltpu.VMEM((CH,), jnp.int32),
            pltpu.VMEM((CH + LANES,), jnp.float32),
            pltpu.VMEM((CH,), jnp.int32),
            pltpu.VMEM((CH, C_IN), jnp.float32),
            pltpu.VMEM((SLAB, C_IN), jnp.float32),
            pltpu.VMEM_SHARED((QHALF, C_IN), jnp.float32),
            pltpu.SemaphoreType.DMA,
        ],
    )
    return k(x2, cols, rows, vals, bounds)


NP = 5
MM_BT = 2048


def _mm_body(p_ref, w_ref, *o_refs):
    res = lax.dot_general(
        p_ref[...], w_ref[...], (((1,), (0,)), ((), ())),
        preferred_element_type=jnp.float32)
    for p5 in range(NP):
        o_refs[p5][...] = res[:, p5 * C_IN:(p5 + 1) * C_IN]


def _matmul(pooled2, w640):
    return pl.pallas_call(
        _mm_body,
        grid=(B * N_HIGH // MM_BT,),
        in_specs=[
            pl.BlockSpec((MM_BT, C_IN), lambda t: (t, 0)),
            pl.BlockSpec((C_IN, NP * C_IN), lambda t: (0, 0)),
        ],
        out_specs=[pl.BlockSpec((MM_BT, C_IN), lambda t: (t, 0))] * NP,
        out_shape=[jax.ShapeDtypeStruct((B * N_HIGH, C_IN), jnp.float32)] * NP,
    )(pooled2, w640)


N_CHUNKS = B * N_GCH


def _spiral_body(h0, h1, h2, h3, h4, idxt_hbm, bias_hbm, out_hbm,
                 biasv, idxfull, idxv, gbuf, ybuf, sem):
    tables = (h0, h1, h2, h3, h4)
    wid = lax.axis_index("s") * NC + lax.axis_index("c")
    nb = wid * ROWS_PER_W
    pltpu.sync_copy(bias_hbm, biasv)
    for j in range(J):
        pltpu.sync_copy(idxt_hbm.at[pl.ds(j * N_HIGH + nb, ROWS_PER_W)],
                        idxfull.at[pl.ds(j * ROWS_PER_W, ROWS_PER_W)])

    def load_idx(t, par):
        b = t // N_GCH
        c = t % N_GCH
        off = b * N_HIGH
        for j in range(J):
            for q in range(GCH // LANES):
                dsl = pl.ds(par * J * GCH + j * GCH + q * LANES, LANES)
                ssl = pl.ds(j * ROWS_PER_W + c * GCH + q * LANES, LANES)
                idxv[dsl] = idxfull[ssl] + off

    def fire(par):
        for j in range(J):
            pltpu.async_copy(
                tables[j // 2].at[
                    idxv.at[pl.ds(par * J * GCH + j * GCH, GCH)]],
                gbuf.at[par, j], sem)

    def drain(par):
        for j in range(J):
            pltpu.make_async_copy(
                tables[j // 2].at[
                    idxv.at[pl.ds(par * J * GCH + j * GCH, GCH)]],
                gbuf.at[par, j], sem).wait()

    def accum_store(t, par):
        b = t // N_GCH
        c = t % N_GCH

        def r_body(r, _):
            acc = [biasv[pl.ds(cb * LANES, LANES)]
                   for cb in range(C_OUT // LANES)]
            for j in range(J):
                for cb in range(C_OUT // LANES):
                    hsl = pl.ds((j % 2) * C_OUT + cb * LANES, LANES)
                    acc[cb] = acc[cb] + gbuf[par, j, r, hsl]
            for cb in range(C_OUT // LANES):
                ybuf[r, pl.ds(cb * LANES, LANES)] = jnp.maximum(acc[cb], 0.0)
            return 0
        lax.fori_loop(0, GCH, r_body, 0)

        pltpu.sync_copy(ybuf, out_hbm.at[b, pl.ds(nb + c * GCH, GCH)])

    load_idx(0, 0)
    fire(0)

    def t2_body(t2, _):
        t = 2 * t2
        drain(0)
        load_idx(t + 1, 1)
        fire(1)
        accum_store(t, 0)
        drain(1)

        @pl.when(t2 < N_CHUNKS // 2 - 1)
        def _():
            load_idx(t + 2, 0)
            fire(0)
        accum_store(t + 1, 1)
        return 0
    lax.fori_loop(0, N_CHUNKS // 2, t2_body, 0)


def _spiral(hs, idxt, bias):
    k = pl.kernel(
        _spiral_body,
        out_type=jax.ShapeDtypeStruct((B, N_HIGH, C_OUT), jnp.float32),
        mesh=_mesh(),
        scratch_types=[
            pltpu.VMEM((C_OUT,), jnp.float32),
            pltpu.VMEM((J * ROWS_PER_W,), jnp.int32),
            pltpu.VMEM((2 * J * GCH,), jnp.int32),
            pltpu.VMEM((2, J, GCH, C_IN), jnp.float32),
            pltpu.VMEM((GCH, C_OUT), jnp.float32),
            pltpu.SemaphoreType.DMA,
        ],
        compiler_params=pltpu.CompilerParams(use_tc_tiling_on_sc=False,
                                             needs_layout_passes=False),
    )
    return k(*hs, idxt, bias)


def kernel(x, up_rows, up_cols, up_vals, indices,
           W1, b1, W_d3, b_d3, W_2d3, b_2d3, W_full, b_full):
    wf = W_full.reshape(C_OUT // 2, J, C_IN).transpose(1, 0, 2)
    w2 = W_2d3.reshape(C_OUT // 4, 6, C_IN).transpose(1, 0, 2)
    w2 = jnp.concatenate([w2, jnp.zeros((3, C_OUT // 4, C_IN), jnp.float32)], 0)
    w3 = W_d3.reshape(C_OUT // 4, 3, C_IN).transpose(1, 0, 2)
    w3 = jnp.concatenate([w3, jnp.zeros((6, C_OUT // 4, C_IN), jnp.float32)], 0)
    wcat = jnp.concatenate([wf, w2, w3], axis=1)
    wcat = wcat.at[0].add(W1)
    bias = jnp.concatenate([b_full, b_2d3, b_d3]) + b1

    edges = jnp.arange(0, N_HIGH + 1, ROWS_PER_W, dtype=jnp.int32)
    bounds = jnp.searchsorted(up_rows, edges).astype(jnp.int32)
    bounds = jnp.concatenate(
        [bounds, jnp.zeros((NW + LANES + 8 - (NW + 1),), jnp.int32)])
    cols_p = jnp.concatenate([up_cols.astype(jnp.int32),
                              jnp.zeros((CH,), jnp.int32)])
    rows_p = jnp.concatenate([up_rows.astype(jnp.int32),
                              jnp.full((CH,), N_HIGH, jnp.int32)])
    vals_p = jnp.concatenate([up_vals, jnp.zeros((CH,), jnp.float32)])

    w640 = jnp.pad(wcat.transpose(2, 0, 1).reshape(C_IN, J * C_OUT),
                   ((0, 0), (0, NP * C_IN - J * C_OUT)))

    x2 = x.reshape(B * N_LOW, C_IN)
    pooled = _pool(x2, cols_p, rows_p, vals_p, bounds)
    pooled2 = pooled.reshape(B * N_HIGH, C_IN)
    hs = _matmul(pooled2, w640)
    idxt = indices.astype(jnp.int32).T.reshape(J * N_HIGH)
    return _spiral(hs, idxt, bias)


# speedup vs baseline: 1.5800x; 1.2269x over previous
import functools

import jax
import jax.numpy as jnp
from jax import lax
from jax.experimental import pallas as pl
from jax.experimental.pallas import tpu as pltpu
from jax.experimental.pallas import tpu_sc as plsc

B = 4
N_LOW = 8192
N_HIGH = 16384
C_IN = 128
C_OUT = 64
J = 9
NNZ = 3 * N_HIGH

NC, NS, LANES = 2, 16, 16
NW = NC * NS
ROWS_PER_W = N_HIGH // NW
CH = 128
GCH = 32
N_GCH = ROWS_PER_W // GCH

_mesh = lambda: plsc.VectorSubcoreMesh(core_axis_name="c", subcore_axis_name="s")


NSEG = 4
QHALF = N_HIGH // NSEG
SLAB = QHALF // NS


NCHT = NNZ // CH


def _pool_body(x2_hbm, meta_hbm, vals_hbm, bounds_hbm, out_hbm,
               boundsv, metaA, metaB, valA, valB, colvA, colvB, idxbA, idxbB,
               xgA, xgB, zbuf, shared, semM, semG, semS):
    cid = lax.axis_index("c")
    sid = lax.axis_index("s")
    pltpu.sync_copy(bounds_hbm, boundsv)
    bufs = ((metaA, valA, colvA, idxbA, xgA), (metaB, valB, colvB, idxbB, xgB))

    def zb_body(r, _):
        for cb in range(C_IN // LANES):
            zbuf[r, pl.ds(cb * LANES, LANES)] = jnp.zeros((LANES,), jnp.float32)
        return 0
    lax.fori_loop(0, SLAB, zb_body, 0)

    def bp_body(t, _):
        b = t // 2
        seg = cid * 2 + (t % 2)
        rbase = seg * QHALF
        s = boundsv[pl.ds(seg * 8, LANES)][0]
        e = boundsv[pl.ds(seg * 8 + 8, LANES)][0]
        c0 = s // CH + sid
        c1 = (e + CH - 1) // CH
        nci = (c1 - c0 + NS - 1) // NS

        pltpu.sync_copy(zbuf, shared.at[pl.ds(sid * SLAB, SLAB)])
        plsc.subcore_barrier()

        col_off = b * N_LOW

        def load_meta(ci, mb, vb, sem=None):
            src = meta_hbm.at[pl.ds(ci * 2 * CH, 2 * CH)]
            dst = mb.at[pl.ds(0, 2 * CH)]
            vsrc = vals_hbm.at[pl.ds(ci * CH, CH)]
            vdst = vb.at[pl.ds(0, CH)]
            if sem is None:
                pltpu.sync_copy(src, dst)
                pltpu.sync_copy(vsrc, vdst)
            else:
                pltpu.async_copy(src, dst, sem)
                pltpu.async_copy(vsrc, vdst, sem)

        def wait_meta(mb, vb):
            pltpu.make_async_copy(meta_hbm.at[pl.ds(0, 2 * CH)],
                                  mb.at[pl.ds(0, 2 * CH)], semM).wait()
            pltpu.make_async_copy(vals_hbm.at[pl.ds(0, CH)],
                                  vb.at[pl.ds(0, CH)], semM).wait()

        def comp_idx(mb, cv, ib):
            for q in range(CH // LANES):
                sl = pl.ds(q * LANES, LANES)
                cv[sl] = mb[sl] + col_off
                ib[sl] = jnp.clip(
                    mb[pl.ds(CH + q * LANES, LANES)] - rbase, 0, QHALF - 1)

        def scale(i, vb, xg):
            off = (c0 + i * NS) * CH

            def r_body(r, _):
                kk = off + r
                v = vb[pl.ds(r, LANES)][0]
                v = jnp.where((kk >= s) & (kk < e), v, 0.0)
                vv = jnp.full((LANES,), v, jnp.float32)
                for cb in range(C_IN // LANES):
                    sl = pl.ds(cb * LANES, LANES)
                    xg[r, sl] = xg[r, sl] * vv
                return 0
            lax.fori_loop(0, CH, r_body, 0)

        def step(i, cur, nxt):
            mb, vb, cv, ib, xg = cur
            mbn, vbn, cvn, ibn, xgn = nxt

            @pl.when(i + 1 < nci)
            def _():
                load_meta(c0 + (i + 1) * NS, mbn, vbn, semM)
            pltpu.make_async_copy(x2_hbm.at[cv], xg, semG).wait()

            @pl.when(i + 1 < nci)
            def _():
                @pl.when(i >= 1)
                def _():
                    pltpu.make_async_copy(xgn, shared.at[ibn], semS).wait()
                wait_meta(mbn, vbn)
                comp_idx(mbn, cvn, ibn)
                pltpu.async_copy(x2_hbm.at[cvn], xgn, semG)

            scale(i, vb, xg)
            pltpu.async_copy(xg, shared.at[ib], semS, add=True)

        @pl.when(nci > 0)
        def _():
            load_meta(c0, metaA, valA)
            comp_idx(metaA, colvA, idxbA)
            pltpu.async_copy(x2_hbm.at[colvA], xgA, semG)

        def pair_body(i2, _):
            i = 2 * i2

            @pl.when(i < nci)
            def _():
                step(i, bufs[0], bufs[1])

            @pl.when(i + 1 < nci)
            def _():
                step(i + 1, bufs[1], bufs[0])
            return 0
        lax.fori_loop(0, (nci + 1) // 2, pair_body, 0)

        @pl.when(nci >= 1)
        def _():
            pltpu.make_async_copy(xgA, shared.at[idxbA], semS).wait()

        @pl.when(nci >= 2)
        def _():
            pltpu.make_async_copy(xgA, shared.at[idxbA], semS).wait()

        plsc.subcore_barrier()
        pltpu.sync_copy(shared.at[pl.ds(sid * SLAB, SLAB)],
                        out_hbm.at[b, pl.ds(rbase + sid * SLAB, SLAB)])
        plsc.subcore_barrier()
        return 0
    lax.fori_loop(0, B * 2, bp_body, 0)


def _pool(x2, meta, vals, bounds):
    k = pl.kernel(
        _pool_body,
        out_type=jax.ShapeDtypeStruct((B, N_HIGH, C_IN), jnp.float32),
        mesh=_mesh(),
        scratch_types=[
            pltpu.VMEM((NW + LANES + 8,), jnp.int32),
            pltpu.VMEM((2 * CH,), jnp.int32),
            pltpu.VMEM((2 * CH,), jnp.int32),
            pltpu.VMEM((CH + LANES,), jnp.float32),
            pltpu.VMEM((CH + LANES,), jnp.float32),
            pltpu.VMEM((CH,), jnp.int32),
            pltpu.VMEM((CH,), jnp.int32),
            pltpu.VMEM((CH,), jnp.int32),
            pltpu.VMEM((CH,), jnp.int32),
            pltpu.VMEM((CH, C_IN), jnp.float32),
            pltpu.VMEM((CH, C_IN), jnp.float32),
            pltpu.VMEM((SLAB, C_IN), jnp.float32),
            pltpu.VMEM_SHARED((QHALF, C_IN), jnp.float32),
            pltpu.SemaphoreType.DMA,
            pltpu.SemaphoreType.DMA,
            pltpu.SemaphoreType.DMA,
        ],
    )
    return k(x2, meta, vals, bounds)


NP = 5
MM_BT = 2048


def _mm_body(p_ref, w_ref, *o_refs):
    res = lax.dot_general(
        p_ref[...], w_ref[...], (((1,), (0,)), ((), ())),
        preferred_element_type=jnp.float32)
    for p5 in range(NP):
        o_refs[p5][...] = res[:, p5 * C_IN:(p5 + 1) * C_IN]


def _matmul(pooled2, w640):
    return pl.pallas_call(
        _mm_body,
        grid=(B * N_HIGH // MM_BT,),
        in_specs=[
            pl.BlockSpec((MM_BT, C_IN), lambda t: (t, 0)),
            pl.BlockSpec((C_IN, NP * C_IN), lambda t: (0, 0)),
        ],
        out_specs=[pl.BlockSpec((MM_BT, C_IN), lambda t: (t, 0))] * NP,
        out_shape=[jax.ShapeDtypeStruct((B * N_HIGH, C_IN), jnp.float32)] * NP,
    )(pooled2, w640)


N_CHUNKS = B * N_GCH


def _spiral_body(h0, h1, h2, h3, h4, idxt_hbm, bias_hbm, out_hbm,
                 biasv, idxfull, idxv, gbuf, ybuf, sem):
    tables = (h0, h1, h2, h3, h4)
    wid = lax.axis_index("s") * NC + lax.axis_index("c")
    nb = wid * ROWS_PER_W
    pltpu.sync_copy(bias_hbm, biasv)
    for j in range(J):
        pltpu.sync_copy(idxt_hbm.at[pl.ds(j * N_HIGH + nb, ROWS_PER_W)],
                        idxfull.at[pl.ds(j * ROWS_PER_W, ROWS_PER_W)])

    def load_idx(t, par):
        b = t // N_GCH
        c = t % N_GCH
        off = b * N_HIGH
        for j in range(J):
            for q in range(GCH // LANES):
                dsl = pl.ds(par * J * GCH + j * GCH + q * LANES, LANES)
                ssl = pl.ds(j * ROWS_PER_W + c * GCH + q * LANES, LANES)
                idxv[dsl] = idxfull[ssl] + off

    def fire(par):
        for j in range(J):
            pltpu.async_copy(
                tables[j // 2].at[
                    idxv.at[pl.ds(par * J * GCH + j * GCH, GCH)]],
                gbuf.at[par, j], sem)

    def drain(par):
        for j in range(J):
            pltpu.make_async_copy(
                tables[j // 2].at[
                    idxv.at[pl.ds(par * J * GCH + j * GCH, GCH)]],
                gbuf.at[par, j], sem).wait()

    def accum_store(t, par):
        b = t // N_GCH
        c = t % N_GCH

        def r_body(r, _):
            acc = [biasv[pl.ds(cb * LANES, LANES)]
                   for cb in range(C_OUT // LANES)]
            for j in range(J):
                for cb in range(C_OUT // LANES):
                    hsl = pl.ds((j % 2) * C_OUT + cb * LANES, LANES)
                    acc[cb] = acc[cb] + gbuf[par, j, r, hsl]
            for cb in range(C_OUT // LANES):
                ybuf[r, pl.ds(cb * LANES, LANES)] = jnp.maximum(acc[cb], 0.0)
            return 0
        lax.fori_loop(0, GCH, r_body, 0)

        pltpu.sync_copy(ybuf, out_hbm.at[b, pl.ds(nb + c * GCH, GCH)])

    load_idx(0, 0)
    fire(0)

    def t2_body(t2, _):
        t = 2 * t2
        drain(0)
        load_idx(t + 1, 1)
        fire(1)
        accum_store(t, 0)
        drain(1)

        @pl.when(t2 < N_CHUNKS // 2 - 1)
        def _():
            load_idx(t + 2, 0)
            fire(0)
        accum_store(t + 1, 1)
        return 0
    lax.fori_loop(0, N_CHUNKS // 2, t2_body, 0)


def _spiral(hs, idxt, bias):
    k = pl.kernel(
        _spiral_body,
        out_type=jax.ShapeDtypeStruct((B, N_HIGH, C_OUT), jnp.float32),
        mesh=_mesh(),
        scratch_types=[
            pltpu.VMEM((C_OUT,), jnp.float32),
            pltpu.VMEM((J * ROWS_PER_W,), jnp.int32),
            pltpu.VMEM((2 * J * GCH,), jnp.int32),
            pltpu.VMEM((2, J, GCH, C_IN), jnp.float32),
            pltpu.VMEM((GCH, C_OUT), jnp.float32),
            pltpu.SemaphoreType.DMA,
        ],
        compiler_params=pltpu.CompilerParams(use_tc_tiling_on_sc=False,
                                             needs_layout_passes=False),
    )
    return k(*hs, idxt, bias)


def kernel(x, up_rows, up_cols, up_vals, indices,
           W1, b1, W_d3, b_d3, W_2d3, b_2d3, W_full, b_full):
    wf = W_full.reshape(C_OUT // 2, J, C_IN).transpose(1, 0, 2)
    w2 = W_2d3.reshape(C_OUT // 4, 6, C_IN).transpose(1, 0, 2)
    w2 = jnp.concatenate([w2, jnp.zeros((3, C_OUT // 4, C_IN), jnp.float32)], 0)
    w3 = W_d3.reshape(C_OUT // 4, 3, C_IN).transpose(1, 0, 2)
    w3 = jnp.concatenate([w3, jnp.zeros((6, C_OUT // 4, C_IN), jnp.float32)], 0)
    wcat = jnp.concatenate([wf, w2, w3], axis=1)
    wcat = wcat.at[0].add(W1)
    bias = jnp.concatenate([b_full, b_2d3, b_d3]) + b1

    edges = jnp.arange(0, N_HIGH + 1, ROWS_PER_W, dtype=jnp.int32)
    bounds = jnp.searchsorted(up_rows, edges).astype(jnp.int32)
    bounds = jnp.concatenate(
        [bounds, jnp.zeros((NW + LANES + 8 - (NW + 1),), jnp.int32)])
    meta = jnp.stack([up_cols.astype(jnp.int32).reshape(NCHT, CH),
                      up_rows.astype(jnp.int32).reshape(NCHT, CH)],
                     axis=1).reshape(NCHT * 2 * CH)

    w640 = jnp.pad(wcat.transpose(2, 0, 1).reshape(C_IN, J * C_OUT),
                   ((0, 0), (0, NP * C_IN - J * C_OUT)))

    x2 = x.reshape(B * N_LOW, C_IN)
    pooled = _pool(x2, meta, up_vals, bounds)
    pooled2 = pooled.reshape(B * N_HIGH, C_IN)
    hs = _matmul(pooled2, w640)
    idxt = indices.astype(jnp.int32).T.reshape(J * N_HIGH)
    return _spiral(hs, idxt, bias)


# speedup vs baseline: 1.6454x; 1.0414x over previous
import functools

import jax
import jax.numpy as jnp
from jax import lax
from jax.experimental import pallas as pl
from jax.experimental.pallas import tpu as pltpu
from jax.experimental.pallas import tpu_sc as plsc

B = 4
N_LOW = 8192
N_HIGH = 16384
C_IN = 128
C_OUT = 64
J = 9
NNZ = 3 * N_HIGH

NC, NS, LANES = 2, 16, 16
NW = NC * NS
ROWS_PER_W = N_HIGH // NW
CH = 128
GCH = 32
N_GCH = ROWS_PER_W // GCH

_mesh = lambda: plsc.VectorSubcoreMesh(core_axis_name="c", subcore_axis_name="s")


NSEG = 4
QHALF = N_HIGH // NSEG
SLAB = QHALF // NS


NCHT = NNZ // CH


def _pool_body(x2_hbm, meta_hbm, vals_hbm, bounds_hbm, out_hbm,
               boundsv, metaA, metaB, valA, valB, colvA, colvB, idxbA, idxbB,
               xgA, xgB, zbuf, shared, semM, semG, semS):
    cid = lax.axis_index("c")
    sid = lax.axis_index("s")
    pltpu.sync_copy(bounds_hbm, boundsv)
    bufs = ((metaA, valA, colvA, idxbA, xgA), (metaB, valB, colvB, idxbB, xgB))

    def zb_body(r, _):
        for cb in range(C_IN // LANES):
            zbuf[r, pl.ds(cb * LANES, LANES)] = jnp.zeros((LANES,), jnp.float32)
        return 0
    lax.fori_loop(0, SLAB, zb_body, 0)

    def bp_body(t, _):
        b = t // 2
        seg = cid * 2 + (t % 2)
        rbase = seg * QHALF
        s = boundsv[pl.ds(seg * 8, LANES)][0]
        e = boundsv[pl.ds(seg * 8 + 8, LANES)][0]
        c0 = s // CH + sid
        c1 = (e + CH - 1) // CH
        nci = (c1 - c0 + NS - 1) // NS

        pltpu.sync_copy(zbuf, shared.at[pl.ds(sid * SLAB, SLAB)])
        plsc.subcore_barrier()

        col_off = b * N_LOW

        def load_meta(ci, mb, vb, sem=None):
            src = meta_hbm.at[pl.ds(ci * 2 * CH, 2 * CH)]
            dst = mb.at[pl.ds(0, 2 * CH)]
            vsrc = vals_hbm.at[pl.ds(ci * CH, CH)]
            vdst = vb.at[pl.ds(0, CH)]
            if sem is None:
                pltpu.sync_copy(src, dst)
                pltpu.sync_copy(vsrc, vdst)
            else:
                pltpu.async_copy(src, dst, sem)
                pltpu.async_copy(vsrc, vdst, sem)

        def wait_meta(mb, vb):
            pltpu.make_async_copy(meta_hbm.at[pl.ds(0, 2 * CH)],
                                  mb.at[pl.ds(0, 2 * CH)], semM).wait()
            pltpu.make_async_copy(vals_hbm.at[pl.ds(0, CH)],
                                  vb.at[pl.ds(0, CH)], semM).wait()

        def comp_idx(mb, cv, ib):
            for q in range(CH // LANES):
                sl = pl.ds(q * LANES, LANES)
                cv[sl] = mb[sl] + col_off
                ib[sl] = jnp.clip(
                    mb[pl.ds(CH + q * LANES, LANES)] - rbase, 0, QHALF - 1)

        def scale(i, vb, xg):
            off = (c0 + i * NS) * CH

            def r_body(r, _):
                kk = off + r
                v = vb[pl.ds(r, LANES)][0]
                v = jnp.where((kk >= s) & (kk < e), v, 0.0)
                vv = jnp.full((LANES,), v, jnp.float32)
                for cb in range(C_IN // LANES):
                    sl = pl.ds(cb * LANES, LANES)
                    xg[r, sl] = xg[r, sl] * vv
                return 0
            lax.fori_loop(0, CH, r_body, 0)

        def step(i, cur, nxt):
            mb, vb, cv, ib, xg = cur
            mbn, vbn, cvn, ibn, xgn = nxt

            @pl.when(i + 1 < nci)
            def _():
                load_meta(c0 + (i + 1) * NS, mbn, vbn, semM)
            pltpu.make_async_copy(x2_hbm.at[cv], xg, semG).wait()

            @pl.when(i + 1 < nci)
            def _():
                @pl.when(i >= 1)
                def _():
                    pltpu.make_async_copy(xgn, shared.at[ibn], semS).wait()
                wait_meta(mbn, vbn)
                comp_idx(mbn, cvn, ibn)
                pltpu.async_copy(x2_hbm.at[cvn], xgn, semG)

            scale(i, vb, xg)
            pltpu.async_copy(xg, shared.at[ib], semS, add=True)

        @pl.when(nci > 0)
        def _():
            load_meta(c0, metaA, valA)
            comp_idx(metaA, colvA, idxbA)
            pltpu.async_copy(x2_hbm.at[colvA], xgA, semG)

        def pair_body(i2, _):
            i = 2 * i2

            @pl.when(i < nci)
            def _():
                step(i, bufs[0], bufs[1])

            @pl.when(i + 1 < nci)
            def _():
                step(i + 1, bufs[1], bufs[0])
            return 0
        lax.fori_loop(0, (nci + 1) // 2, pair_body, 0)

        @pl.when(nci >= 1)
        def _():
            pltpu.make_async_copy(xgA, shared.at[idxbA], semS).wait()

        @pl.when(nci >= 2)
        def _():
            pltpu.make_async_copy(xgA, shared.at[idxbA], semS).wait()

        plsc.subcore_barrier()
        pltpu.sync_copy(shared.at[pl.ds(sid * SLAB, SLAB)],
                        out_hbm.at[b, pl.ds(rbase + sid * SLAB, SLAB)])
        plsc.subcore_barrier()
        return 0
    lax.fori_loop(0, B * 2, bp_body, 0)


def _pool(x2, meta, vals, bounds):
    k = pl.kernel(
        _pool_body,
        out_type=jax.ShapeDtypeStruct((B, N_HIGH, C_IN), jnp.float32),
        mesh=_mesh(),
        scratch_types=[
            pltpu.VMEM((NW + LANES + 8,), jnp.int32),
            pltpu.VMEM((2 * CH,), jnp.int32),
            pltpu.VMEM((2 * CH,), jnp.int32),
            pltpu.VMEM((CH + LANES,), jnp.float32),
            pltpu.VMEM((CH + LANES,), jnp.float32),
            pltpu.VMEM((CH,), jnp.int32),
            pltpu.VMEM((CH,), jnp.int32),
            pltpu.VMEM((CH,), jnp.int32),
            pltpu.VMEM((CH,), jnp.int32),
            pltpu.VMEM((CH, C_IN), jnp.float32),
            pltpu.VMEM((CH, C_IN), jnp.float32),
            pltpu.VMEM((SLAB, C_IN), jnp.float32),
            pltpu.VMEM_SHARED((QHALF, C_IN), jnp.float32),
            pltpu.SemaphoreType.DMA,
            pltpu.SemaphoreType.DMA,
            pltpu.SemaphoreType.DMA,
        ],
    )
    return k(x2, meta, vals, bounds)


NP = 5
MM_BT = 2048


def _mm_body(p_ref, w_ref, *o_refs):
    res = lax.dot_general(
        p_ref[...], w_ref[...], (((1,), (0,)), ((), ())),
        preferred_element_type=jnp.float32)
    for p5 in range(NP):
        o_refs[p5][...] = res[:, p5 * C_IN:(p5 + 1) * C_IN]


def _matmul(pooled2, w640):
    return pl.pallas_call(
        _mm_body,
        grid=(B * N_HIGH // MM_BT,),
        in_specs=[
            pl.BlockSpec((MM_BT, C_IN), lambda t: (t, 0)),
            pl.BlockSpec((C_IN, NP * C_IN), lambda t: (0, 0)),
        ],
        out_specs=[pl.BlockSpec((MM_BT, C_IN), lambda t: (t, 0))] * NP,
        out_shape=[jax.ShapeDtypeStruct((B * N_HIGH, C_IN), jnp.float32)] * NP,
    )(pooled2, w640)


N_CHUNKS = B * N_GCH


def _spiral_body(h0, h1, h2, h3, h4, idxt_hbm, bias_hbm, out_hbm,
                 biasv, idxfull, idxv, gbuf, ybuf, sem):
    tables = (h0, h1, h2, h3, h4)
    wid = lax.axis_index("s") * NC + lax.axis_index("c")
    nb = wid * ROWS_PER_W
    pltpu.sync_copy(bias_hbm, biasv)
    for j in range(J):
        pltpu.sync_copy(idxt_hbm.at[pl.ds(j * N_HIGH + nb, ROWS_PER_W)],
                        idxfull.at[pl.ds(j * ROWS_PER_W, ROWS_PER_W)])

    def load_idx(t, par):
        b = t // N_GCH
        c = t % N_GCH
        off = b * N_HIGH
        for j in range(J):
            for q in range(GCH // LANES):
                dsl = pl.ds(par * J * GCH + j * GCH + q * LANES, LANES)
                ssl = pl.ds(j * ROWS_PER_W + c * GCH + q * LANES, LANES)
                idxv[dsl] = idxfull[ssl] + off

    def fire(par):
        for j in range(J):
            pltpu.async_copy(
                tables[j // 2].at[
                    idxv.at[pl.ds(par * J * GCH + j * GCH, GCH)]],
                gbuf.at[par, j], sem)

    def drain(par):
        for j in range(J):
            pltpu.make_async_copy(
                tables[j // 2].at[
                    idxv.at[pl.ds(par * J * GCH + j * GCH, GCH)]],
                gbuf.at[par, j], sem).wait()

    def accum_store(t, par):
        b = t // N_GCH
        c = t % N_GCH

        def r_body(r, _):
            acc = [biasv[pl.ds(cb * LANES, LANES)]
                   for cb in range(C_OUT // LANES)]
            for j in range(J):
                for cb in range(C_OUT // LANES):
                    hsl = pl.ds((j % 2) * C_OUT + cb * LANES, LANES)
                    acc[cb] = acc[cb] + gbuf[par, j, r, hsl]
            for cb in range(C_OUT // LANES):
                ybuf[r, pl.ds(cb * LANES, LANES)] = jnp.maximum(acc[cb], 0.0)
            return 0
        lax.fori_loop(0, GCH, r_body, 0)

        pltpu.sync_copy(ybuf, out_hbm.at[b, pl.ds(nb + c * GCH, GCH)])

    load_idx(0, 0)
    fire(0)

    def t2_body(t2, _):
        t = 2 * t2
        drain(0)
        load_idx(t + 1, 1)
        fire(1)
        accum_store(t, 0)
        drain(1)

        @pl.when(t2 < N_CHUNKS // 2 - 1)
        def _():
            load_idx(t + 2, 0)
            fire(0)
        accum_store(t + 1, 1)
        return 0
    lax.fori_loop(0, N_CHUNKS // 2, t2_body, 0)


def _spiral(hs, idxt, bias):
    k = pl.kernel(
        _spiral_body,
        out_type=jax.ShapeDtypeStruct((B, N_HIGH, C_OUT), jnp.float32),
        mesh=_mesh(),
        scratch_types=[
            pltpu.VMEM((C_OUT,), jnp.float32),
            pltpu.VMEM((J * ROWS_PER_W,), jnp.int32),
            pltpu.VMEM((2 * J * GCH,), jnp.int32),
            pltpu.VMEM((2, J, GCH, C_IN), jnp.float32),
            pltpu.VMEM((GCH, C_OUT), jnp.float32),
            pltpu.SemaphoreType.DMA,
        ],
        compiler_params=pltpu.CompilerParams(use_tc_tiling_on_sc=False,
                                             needs_layout_passes=False),
    )
    return k(*hs, idxt, bias)


def kernel(x, up_rows, up_cols, up_vals, indices,
           W1, b1, W_d3, b_d3, W_2d3, b_2d3, W_full, b_full):
    wf = W_full.reshape(C_OUT // 2, J, C_IN).transpose(1, 0, 2)
    w2 = W_2d3.reshape(C_OUT // 4, 6, C_IN).transpose(1, 0, 2)
    w2 = jnp.concatenate([w2, jnp.zeros((3, C_OUT // 4, C_IN), jnp.float32)], 0)
    w3 = W_d3.reshape(C_OUT // 4, 3, C_IN).transpose(1, 0, 2)
    w3 = jnp.concatenate([w3, jnp.zeros((6, C_OUT // 4, C_IN), jnp.float32)], 0)
    wcat = jnp.concatenate([wf, w2, w3], axis=1)
    wcat = wcat.at[0].add(W1)
    bias = jnp.concatenate([b_full, b_2d3, b_d3]) + b1

    edges = jnp.arange(0, N_HIGH + 1, ROWS_PER_W, dtype=jnp.int32)
    bounds = jnp.sum(up_rows[None, :] < edges[:, None],
                     axis=1, dtype=jnp.int32)
    bounds = jnp.concatenate(
        [bounds, jnp.zeros((NW + LANES + 8 - (NW + 1),), jnp.int32)])
    meta = jnp.stack([up_cols.astype(jnp.int32).reshape(NCHT, CH),
                      up_rows.astype(jnp.int32).reshape(NCHT, CH)],
                     axis=1).reshape(NCHT * 2 * CH)

    w640 = jnp.pad(wcat.transpose(2, 0, 1).reshape(C_IN, J * C_OUT),
                   ((0, 0), (0, NP * C_IN - J * C_OUT)))

    x2 = x.reshape(B * N_LOW, C_IN)
    pooled = _pool(x2, meta, up_vals, bounds)
    pooled2 = pooled.reshape(B * N_HIGH, C_IN)
    hs = _matmul(pooled2, w640)
    idxt = indices.astype(jnp.int32).T.reshape(J * N_HIGH)
    return _spiral(hs, idxt, bias)
